# baseline (speedup 1.0000x reference)
import functools

import jax
import jax.numpy as jnp
from jax import lax
from jax.experimental import pallas as pl
from jax.experimental.pallas import tpu as pltpu
from jax.experimental.pallas import tpu_sc as plsc

K = 4
VB = 8
NC = 2
NS = 16
NW = NC * NS


def _topk_core(text, vidT):
    F, vb, D = vidT.shape
    T = text.shape[0]
    sims = jax.lax.dot_general(
        vidT.reshape(F * vb, D), text,
        (((1,), (1,)), ((), ())),
        preferred_element_type=jnp.float32,
    ).reshape(F, vb, T)

    f_iota = jax.lax.broadcasted_iota(jnp.int32, (F, vb, T), 0)
    cur = sims
    idxs = []
    for j in range(K):
        m = jnp.max(cur, axis=0, keepdims=True)
        idx_j = jnp.min(jnp.where(cur == m, f_iota, F), axis=0)
        idxs.append(idx_j)
        cur = jnp.where(f_iota == idx_j[None], -jnp.inf, cur)
    idx = jnp.stack(idxs, axis=1)
    return idx


def _tc1_body(text_ref, vidT_ref, idx_ref):
    idx_ref[...] = _topk_core(text_ref[...], vidT_ref[...])


def _tc2_body(vidT_ref, idx_ref, pooled_ref):
    vidT = vidT_ref[...]
    idx = idx_ref[...]
    F, vb, D = vidT.shape
    T = idx.shape[2]
    f_iota = jax.lax.broadcasted_iota(jnp.int32, (F, vb, T), 0)
    oh = jnp.zeros((F, vb, T), jnp.float32)
    for j in range(K):
        oh = oh + (f_iota == idx[:, j, :][None]).astype(jnp.float32)
    for v in range(vb):
        pooled_ref[v] = jax.lax.dot_general(
            oh[:, v, :], vidT[:, v, :], (((0,), (0,)), ((), ())),
            preferred_element_type=jnp.float32,
        )


def _make_sc_expand(V, T, D):
    VPW = V // NW
    RB = 16
    mesh = plsc.VectorSubcoreMesh(core_axis_name="c", subcore_axis_name="s")

    @functools.partial(
        pl.kernel, mesh=mesh,
        out_type=jax.ShapeDtypeStruct((V, K, D, T), jnp.int32),
        scratch_types=[
            pltpu.VMEM((K, T), jnp.int32),
            pltpu.VMEM((RB, T), jnp.int32),
            pltpu.SemaphoreType.DMA,
        ],
    )
    def expand(idx_hbm, out_hbm, row_v, rep_v, sem):
        wid = lax.axis_index("s") * NC + lax.axis_index("c")
        for v in range(VPW):
            v_abs = wid * VPW + v
            pltpu.sync_copy(idx_hbm.at[v_abs], row_v)
            for j in range(K):
                for c in range(T // 16):
                    val = row_v[j, pl.ds(c * 16, 16)]
                    for r in range(RB):
                        rep_v[r, pl.ds(c * 16, 16)] = val
                cps = [
                    pltpu.async_copy(
                        rep_v, out_hbm.at[v_abs, j, pl.ds(r * RB, RB), :], sem)
                    for r in range(D // RB)
                ]
                for cp in cps:
                    cp.wait()

    return expand


@jax.jit
def kernel(text_embeds, video_embeds):
    T, D = text_embeds.shape
    V, F, _ = video_embeds.shape
    vidT = jnp.transpose(video_embeds, (1, 0, 2))
    grid = (V // VB,)

    idx = pl.pallas_call(
        _tc1_body,
        grid=grid,
        in_specs=[
            pl.BlockSpec((T, D), lambda i: (0, 0)),
            pl.BlockSpec((F, VB, D), lambda i: (0, i, 0)),
        ],
        out_specs=pl.BlockSpec((VB, K, T), lambda i: (i, 0, 0)),
        out_shape=jax.ShapeDtypeStruct((V, K, T), jnp.int32),
    )(text_embeds, vidT)

    idx_exp = _make_sc_expand(V, T, D)(idx)

    pooled = pl.pallas_call(
        _tc2_body,
        grid=grid,
        in_specs=[
            pl.BlockSpec((F, VB, D), lambda i: (0, i, 0)),
            pl.BlockSpec((VB, K, T), lambda i: (i, 0, 0)),
        ],
        out_specs=pl.BlockSpec((VB, T, D), lambda i: (i, 0, 0)),
        out_shape=jax.ShapeDtypeStruct((V, T, D), jnp.float32),
    )(vidT, idx)

    return pooled, idx_exp


# speedup vs baseline: 2.1360x; 2.1360x over previous
import jax
import jax.numpy as jnp
from jax.experimental import pallas as pl

K = 4
VB = 8


def _tc_body(text_ref, vidT_ref, pooled_ref, idx_ref):
    text = text_ref[...]
    vidT = vidT_ref[...]
    F, vb, D = vidT.shape
    T = text.shape[0]

    sims = jax.lax.dot_general(
        vidT.reshape(F * vb, D), text,
        (((1,), (1,)), ((), ())),
        preferred_element_type=jnp.float32,
    ).reshape(F, vb, T)

    f_iota = jax.lax.broadcasted_iota(jnp.int32, (F, vb, T), 0)
    cur = sims
    idxs = []
    for j in range(K):
        m = jnp.max(cur, axis=0, keepdims=True)
        idx_j = jnp.min(jnp.where(cur == m, f_iota, F), axis=0)
        idxs.append(idx_j)
        cur = jnp.where(f_iota == idx_j[None], -jnp.inf, cur)

    idx = jnp.stack(idxs, axis=1)
    idx_ref[...] = jnp.broadcast_to(idx[:, :, None, :], (vb, K, D, T))

    oh = (cur == -jnp.inf).astype(jnp.float32)
    for v in range(vb):
        pooled_ref[v] = jax.lax.dot_general(
            oh[:, v, :], vidT[:, v, :], (((0,), (0,)), ((), ())),
            preferred_element_type=jnp.float32,
        )


@jax.jit
def kernel(text_embeds, video_embeds):
    T, D = text_embeds.shape
    V, F, _ = video_embeds.shape
    grid = (V // VB,)
    pooled, idx_exp = pl.pallas_call(
        _tc_body,
        grid=grid,
        in_specs=[
            pl.BlockSpec((T, D), lambda i: (0, 0)),
            pl.BlockSpec((F, VB, D), lambda i: (0, i, 0)),
        ],
        out_specs=[
            pl.BlockSpec((VB, T, D), lambda i: (i, 0, 0)),
            pl.BlockSpec((VB, K, D, T), lambda i: (i, 0, 0, 0)),
        ],
        out_shape=[
            jax.ShapeDtypeStruct((V, T, D), jnp.float32),
            jax.ShapeDtypeStruct((V, K, D, T), jnp.int32),
        ],
    )(text_embeds, jnp.transpose(video_embeds, (1, 0, 2)))
    return pooled, idx_exp
